# baseline (speedup 1.0000x reference)
import jax
import jax.numpy as jnp
from jax import lax
from jax.experimental import pallas as pl
from jax.experimental.pallas import tpu as pltpu
from jax.experimental.pallas import tpu_sc as plsc

_GAMMA = 2
_A0 = 0.25
_A1 = 0.75
_K = 39
_H = 512
_W = 512
_N = _H * _W
_B = 16
_BH = 16
_Q = _N // 2
_CH = 4096
_NCH = _Q // _CH
_NG = _CH // 16
_NEG = -3.0e38
_POS = 3.0e38
_BIGI = 2**30


def _shift_rows(a, d):
    z = jnp.zeros((abs(d), a.shape[1]), a.dtype)
    if d > 0:
        return jnp.concatenate([a[d:, :], z], axis=0)
    return jnp.concatenate([z, a[:d, :]], axis=0)


def _shift_cols(a, d):
    z = jnp.zeros((a.shape[0], abs(d)), a.dtype)
    if d > 0:
        return jnp.concatenate([a[:, d:], z], axis=1)
    return jnp.concatenate([z, a[:, :d]], axis=1)


def _dense_body(x_ref, t_ref, base_ref, lp_ref, ct_ref):
    b = pl.program_id(0)
    x = x_ref[0, 0]
    t = t_ref[0, 0]

    rs = t
    for d in (1, 2, -1, -2):
        rs = rs + _shift_rows(t, d)
    cs = rs
    for d in (1, 2, -1, -2):
        cs = cs + _shift_cols(rs, d)
    prot = cs > 0.0

    s = jnp.log1p(jnp.exp(-jnp.abs(x)))
    relu = jnp.maximum(x, 0.0)
    logpt = jnp.minimum(x, 0.0) - s
    logpt_bk = -relu - s
    pt = jnp.exp(logpt)
    pt_bk = 1.0 - jnp.exp(logpt_bk)
    focal_pos = -_A1 * (1.0 - pt) ** _GAMMA * logpt
    focal_neg = -_A0 * pt_bk ** _GAMMA * logpt_bk
    base = jnp.sum(jnp.where(t > 0.0, focal_pos, 0.0))

    bce = relu - x * t + s
    lp_ref[...] = jnp.reshape(jnp.where(prot, 0.0, bce), (_N,))
    ct_ref[...] = jnp.reshape(jnp.where(t > 0.0, 0.0, focal_neg), (_N,))

    @pl.when(b == 0)
    def _():
        base_ref[0, 0] = 0.0

    base_ref[0, 0] += base


def _dense(input, target):
    return pl.pallas_call(
        _dense_body,
        grid=(_BH,),
        in_specs=[
            pl.BlockSpec((1, 1, _H, _W), lambda b: (b, 0, 0, 0)),
            pl.BlockSpec((1, 1, _H, _W), lambda b: (b, 0, 0, 0)),
        ],
        out_specs=[
            pl.BlockSpec((1, 1), lambda b: (0, 0), memory_space=pltpu.SMEM),
            pl.BlockSpec((_N,), lambda b: (b,)),
            pl.BlockSpec((_N,), lambda b: (b,)),
        ],
        out_shape=[
            jax.ShapeDtypeStruct((1, 1), jnp.float32),
            jax.ShapeDtypeStruct((_BH * _N,), jnp.float32),
            jax.ShapeDtypeStruct((_BH * _N,), jnp.float32),
        ],
    )(input, target)


def _lane_shuffle_reduce(v, op):
    i0 = lax.broadcasted_iota(jnp.int32, (16,), 0)
    for sh in (8, 4, 2, 1):
        perm = jnp.bitwise_xor(i0, sh)
        v = op(v, v.at[perm].get(mode="promise_in_bounds"))
    return v


def _rmax(v):
    return _lane_shuffle_reduce(v, jnp.maximum)[0]


def _rmin(v):
    return _lane_shuffle_reduce(v, jnp.minimum)[0]


def _rsum(v):
    return _lane_shuffle_reduce(v, jnp.add)[0]


def _sc_body(loss_hbm, ct_hbm, out_hbm,
             buf0, buf1, vals_v, idxs_v, rowi_v, crows_v,
             orow_v, pv_v, pi_v, shv_sp, shi_sp, sem0, sem1, thr_s, evi_s):
    c = lax.axis_index("c")
    s = lax.axis_index("s")
    img = c * (_BH // 2) + lax.shift_right_logical(s, 1)
    half = jnp.bitwise_and(s, 1)
    lio = lax.broadcasted_iota(jnp.int32, (16,), 0)

    gbase = img * _N
    hoff = half * _Q

    def dma(off, buf, sem):
        return pltpu.make_async_copy(
            loss_hbm.at[pl.ds(gbase + off, _CH)], buf.at[pl.ds(0, _CH)], sem)

    def insert(v, i):
        hit = (v > thr_s[0]) | ((v == thr_s[0]) & (i < evi_s[0]))

        @pl.when(hit)
        def _():
            mm = thr_s[0]
            sel = evi_s[0]
            nv, ni = [], []
            for k in range(3):
                va = vals_v[pl.ds(16 * k, 16)]
                ia = idxs_v[pl.ds(16 * k, 16)]
                mk = (va == mm) & (ia == sel)
                nv.append(jnp.where(mk, v, va))
                ni.append(jnp.where(mk, i, ia))
                vals_v[pl.ds(16 * k, 16)] = nv[k]
                idxs_v[pl.ds(16 * k, 16)] = ni[k]
            mm2 = _rmin(jnp.minimum(jnp.minimum(nv[0], nv[1]), nv[2]))
            sel2 = _rmax(jnp.maximum(
                jnp.maximum(jnp.where(nv[0] == mm2, ni[0], -1),
                            jnp.where(nv[1] == mm2, ni[1], -1)),
                jnp.where(nv[2] == mm2, ni[2], -1)))
            thr_s[0] = mm2
            evi_s[0] = sel2

    def publish():
        pltpu.sync_copy(vals_v, shv_sp.at[s])
        pltpu.sync_copy(idxs_v, shi_sp.at[s])

    def merge_from(row):
        pltpu.sync_copy(shv_sp.at[row], pv_v.at[pl.ds(0, 48)])
        pltpu.sync_copy(shi_sp.at[row], pi_v.at[pl.ds(0, 48)])

        for k in range(3):
            pvk = pv_v[pl.ds(16 * k, 16)]
            pik = pi_v[pl.ds(16 * k, 16)]
            for j in range(16):
                if 16 * k + j < _K:
                    insert(pvk[j], pik[j])

    def process(buf, off):
        def g16(i, vms):
            gb = i * 256
            vms = list(vms)
            for j in range(16):
                vms[j % 4] = jnp.maximum(
                    vms[j % 4], buf[pl.ds(gb + j * 16, 16)])
            return tuple(vms)
        init = tuple(jnp.full((16,), _NEG, jnp.float32) for _ in range(4))
        vms = lax.fori_loop(0, _NG // 16, g16, init)
        vm = jnp.maximum(jnp.maximum(vms[0], vms[1]),
                         jnp.maximum(vms[2], vms[3]))
        cmax = _rmax(vm)
        chit = (cmax > thr_s[0]) | ((cmax == thr_s[0]) & (off <= evi_s[0]))

        @pl.when(chit)
        def _():
            def grp(gi, _):
                gb = gi * 16
                v = buf[pl.ds(gb, 16)]
                gm = _rmax(v)
                ghit = ((gm > thr_s[0])
                        | ((gm == thr_s[0]) & (off + gb <= evi_s[0])))

                @pl.when(ghit)
                def _():
                    for j in range(16):
                        insert(v[j], off + gb + j)
                return 0
            lax.fori_loop(0, _NG, grp, 0)

    for k in range(3):
        active = (lio + 16 * k) < _K
        slot = lio + 16 * k
        vals_v[pl.ds(16 * k, 16)] = jnp.where(active, _NEG, _POS)
        idxs_v[pl.ds(16 * k, 16)] = jnp.where(active, _BIGI + slot, 0)
    thr_s[0] = _NEG
    evi_s[0] = _BIGI + _K - 1

    dma(hoff, buf0, sem0).start()
    dma(hoff + _CH, buf1, sem1).start()

    def pair(m, _):
        o0 = hoff + (2 * m) * _CH
        dma(o0, buf0, sem0).wait()
        process(buf0, o0)

        @pl.when(2 * m + 2 < _NCH)
        def _():
            dma(o0 + 2 * _CH, buf0, sem0).start()

        o1 = o0 + _CH
        dma(o1, buf1, sem1).wait()
        process(buf1, o1)

        @pl.when(2 * m + 3 < _NCH)
        def _():
            dma(o1 + 2 * _CH, buf1, sem1).start()
        return 0

    lax.fori_loop(0, _NCH // 2, pair, 0)

    publish()
    plsc.subcore_barrier()

    @pl.when(half == 0)
    def _():
        merge_from(s + 1)

        cols, offs = [], []
        for k in range(3):
            idx = idxs_v[pl.ds(16 * k, 16)]
            gix = gbase + idx
            cols.append(jnp.bitwise_and(idx, 15))
            offs.append(jnp.bitwise_and(gix, -16))

        def line(sj):
            off = pl.multiple_of(offs[sj // 16][sj % 16], 16)
            return pltpu.make_async_copy(
                ct_hbm.at[pl.ds(off, 16)], crows_v.at[sj], sem0)

        for sj in range(_K):
            line(sj).start()
        for sj in range(_K):
            line(sj).wait()

        acc = jnp.zeros((16,), jnp.float32)
        for k in range(3):
            for j in range(16):
                sj = 16 * k + j
                if sj < _K:
                    row = crows_v[sj]
                    acc = acc + jnp.where(lio == cols[k][j], row, 0.0)
        total = _rsum(acc)

        orow_v[...] = jnp.where(lio == 0, total, 0.0)
        pltpu.sync_copy(orow_v, out_hbm.at[img])


def _sc_topk(loss1, ct1):
    mesh = plsc.VectorSubcoreMesh(core_axis_name="c", subcore_axis_name="s")
    return pl.kernel(
        _sc_body,
        out_type=jax.ShapeDtypeStruct((_BH, 16), jnp.float32),
        mesh=mesh,
        scratch_types=[
            pltpu.VMEM((_CH + 16,), jnp.float32),
            pltpu.VMEM((_CH + 16,), jnp.float32),
            pltpu.VMEM((48,), jnp.float32),
            pltpu.VMEM((48,), jnp.int32),
            pltpu.VMEM((48,), jnp.int32),
            pltpu.VMEM((48, 16), jnp.float32),
            pltpu.VMEM((16,), jnp.float32),
            pltpu.VMEM((64,), jnp.float32),
            pltpu.VMEM((64,), jnp.int32),
            pltpu.VMEM_SHARED((16, 48), jnp.float32),
            pltpu.VMEM_SHARED((16, 48), jnp.int32),
            pltpu.SemaphoreType.DMA,
            pltpu.SemaphoreType.DMA,
            pltpu.SMEM((1,), jnp.float32),
            pltpu.SMEM((1,), jnp.int32),
        ],
    )(loss1, ct1)


def kernel(input, target):
    base, lp1, ct1 = _dense(input, target)
    part = _sc_topk(lp1, ct1)
    return base[0, 0] + jnp.sum(part[:, 0])
